# baseline (speedup 1.0000x reference)
import functools

import jax
import jax.numpy as jnp
from jax import lax
from jax.experimental import pallas as pl
from jax.experimental.pallas import tpu as pltpu
from jax.experimental.pallas import tpu_sc as plsc

_B, _S, _D = 64, 512, 768
_EPS = 1e-12

_NC, _NS = 2, 16
_NW = _NC * _NS
_TOK = _B * _S
_SLICES_B = (16, 16, 16, 16)
_LNB = _S
_NSLICE = len(_SLICES_B)
_CH = 32
_NBUF = 4


@functools.lru_cache(maxsize=None)
def _make_sc_gather(per_w):
    nch = per_w // _CH
    slice_tok = per_w * _NW

    def body(ids_hbm, table_hbm, out_hbm, idx_v, rows_v, *sems):
        wid = lax.axis_index("s") * _NC + lax.axis_index("c")
        base = wid * per_w
        pltpu.sync_copy(ids_hbm.at[wid], idx_v)

        gsems = sems[:_NBUF]
        wsems = sems[_NBUF:]

        def fire(c, b):
            return pltpu.async_copy(table_hbm.at[idx_v.at[c]], rows_v.at[b],
                                    gsems[b])

        def drain(c, b):
            return pltpu.async_copy(rows_v.at[b],
                                    out_hbm.at[pl.ds(base + c * _CH, _CH)],
                                    wsems[b])

        gathers = [fire(b, b) for b in range(min(_NBUF, nch))]
        drains = [None] * _NBUF
        for c in range(nch):
            b = c % _NBUF
            gathers[b].wait()
            drains[b] = drain(c, b)
            r = c + 2
            if c >= 2 and r < nch:
                rb = r % _NBUF
                drains[rb].wait()
                gathers[rb] = fire(r, rb)
        for b in range(min(_NBUF, nch)):
            drains[b].wait()

    return pl.kernel(
        body,
        out_type=jax.ShapeDtypeStruct((slice_tok, _D), jnp.float32),
        mesh=plsc.VectorSubcoreMesh(
            core_axis_name="c", subcore_axis_name="s",
            num_cores=_NC, num_subcores=_NS),
        scratch_types=[
            pltpu.VMEM((nch, _CH), jnp.int32),
            pltpu.VMEM((_NBUF, _CH, _D), jnp.float32),
        ] + [pltpu.SemaphoreType.DMA] * (2 * _NBUF),
    )


def _tc_ln_body(x_ref, tt_ref, pos_ref, type_ref, g_ref, b_ref, o_ref):
    x = x_ref[...] + pos_ref[...]
    t0 = type_ref[0:1, :]
    t1 = type_ref[1:2, :]
    tt = tt_ref[...]
    x = x + t0 + tt * (t1 - t0)
    mu = jnp.mean(x, axis=1, keepdims=True)
    xc = x - mu
    var = jnp.mean(xc * xc, axis=1, keepdims=True)
    inv = lax.rsqrt(var + _EPS)
    o_ref[...] = xc * inv * g_ref[...] + b_ref[...]


def _tc_ln_alias_body(x_ref, tt_ref, pos_ref, type_ref, g_ref, b_ref,
                      acc_ref, o_ref):
    del acc_ref
    _tc_ln_body(x_ref, tt_ref, pos_ref, type_ref, g_ref, b_ref, o_ref)


def _tc_ln_slice(start_b, slice_b, gathered, tt, pos_emb, type_emb,
                 gamma2d, beta2d, acc):
    rows_per_blk = _LNB // _S
    base_specs = [
        pl.BlockSpec((_LNB, _D), lambda b: (b, 0)),
        pl.BlockSpec((_LNB, 1), lambda b: (b, 0)),
        pl.BlockSpec((_LNB, _D), lambda b: (0, 0)),
        pl.BlockSpec((2, _D), lambda b: (0, 0)),
        pl.BlockSpec((1, _D), lambda b: (0, 0)),
        pl.BlockSpec((1, _D), lambda b: (0, 0)),
    ]
    out_spec = pl.BlockSpec(
        (_LNB, _D), lambda b: (start_b // rows_per_blk + b, 0))
    grid = (slice_b // rows_per_blk,)
    args = (gathered, tt, pos_emb, type_emb, gamma2d, beta2d)
    if acc is None:
        return pl.pallas_call(
            _tc_ln_body,
            grid=grid,
            in_specs=base_specs,
            out_specs=out_spec,
            out_shape=jax.ShapeDtypeStruct((_TOK, _D), jnp.float32),
        )(*args)
    return pl.pallas_call(
        _tc_ln_alias_body,
        grid=grid,
        in_specs=base_specs + [pl.BlockSpec((8, 128), lambda b: (0, 0))],
        out_specs=out_spec,
        out_shape=jax.ShapeDtypeStruct((_TOK, _D), jnp.float32),
        input_output_aliases={6: 0},
    )(*args, acc)


def kernel(input_ids, token_type_ids, word_emb, pos_emb, type_emb, gamma, beta):
    ids_flat = input_ids.astype(jnp.int32).reshape(_TOK)
    tt_flat = token_type_ids.reshape(_TOK, 1).astype(jnp.float32)
    pos2 = jnp.concatenate([pos_emb] * (_LNB // _S), axis=0)
    gamma2d = gamma.reshape(1, _D)
    beta2d = beta.reshape(1, _D)
    out = None
    start_b = 0
    for slice_b in _SLICES_B:
        stok = slice_b * _S
        per_w = stok // _NW
        ids_k = lax.dynamic_slice_in_dim(ids_flat, start_b * _S, stok)
        tt_k = lax.dynamic_slice_in_dim(tt_flat, start_b * _S, stok)
        gathered = _make_sc_gather(per_w)(
            ids_k.reshape(_NW, per_w // _CH, _CH), word_emb)
        out = _tc_ln_slice(start_b, slice_b, gathered, tt_k, pos2,
                           type_emb, gamma2d, beta2d, out)
        start_b += slice_b
    return out.reshape(_B, _S, _D)


# speedup vs baseline: 1.0575x; 1.0575x over previous
import functools

import jax
import jax.numpy as jnp
from jax import lax
from jax.experimental import pallas as pl
from jax.experimental.pallas import tpu as pltpu
from jax.experimental.pallas import tpu_sc as plsc

_B, _S, _D = 64, 512, 768
_EPS = 1e-12

_NC, _NS = 2, 16
_NW = _NC * _NS
_TOK = _B * _S
_NSLICE = 4
_SLICE_TOK = _TOK // _NSLICE
_SLICE_B = _B // _NSLICE
_PER_W = _SLICE_TOK // _NW
_CH = 32
_NCH = _PER_W // _CH
_NBUF = 4


def _sc_gather_body(ids_hbm, table_hbm, out_hbm, idx_v, rows_v, *sems):
    wid = lax.axis_index("s") * _NC + lax.axis_index("c")
    base = wid * _PER_W
    pltpu.sync_copy(ids_hbm.at[wid], idx_v)

    gsems = sems[:_NBUF]
    wsems = sems[_NBUF:]

    def fire(c, b):
        return pltpu.async_copy(table_hbm.at[idx_v.at[c]], rows_v.at[b],
                                gsems[b])

    def drain(c, b):
        return pltpu.async_copy(rows_v.at[b],
                                out_hbm.at[pl.ds(base + c * _CH, _CH)],
                                wsems[b])

    gathers = [fire(b, b) for b in range(_NBUF)]
    drains = [None] * _NBUF
    for c in range(_NCH):
        b = c % _NBUF
        gathers[b].wait()
        drains[b] = drain(c, b)
        r = c + 2
        if c >= 2 and r < _NCH:
            rb = r % _NBUF
            drains[rb].wait()
            gathers[rb] = fire(r, rb)
    for b in range(_NBUF):
        drains[b].wait()


_sc_gather = functools.partial(
    pl.kernel,
    out_type=jax.ShapeDtypeStruct((_SLICE_TOK, _D), jnp.float32),
    mesh=plsc.VectorSubcoreMesh(
        core_axis_name="c", subcore_axis_name="s",
        num_cores=_NC, num_subcores=_NS),
    scratch_types=[
        pltpu.VMEM((_NCH, _CH), jnp.int32),
        pltpu.VMEM((_NBUF, _CH, _D), jnp.float32),
    ] + [pltpu.SemaphoreType.DMA] * (2 * _NBUF),
)(_sc_gather_body)


def _tc_ln_body(x_ref, tt_ref, pos_ref, type_ref, g_ref, b_ref, o_ref):
    x = x_ref[...] + pos_ref[...]
    t0 = type_ref[0:1, :]
    t1 = type_ref[1:2, :]
    tt = tt_ref[...]
    x = x + t0 + tt * (t1 - t0)
    mu = jnp.mean(x, axis=1, keepdims=True)
    xc = x - mu
    var = jnp.mean(xc * xc, axis=1, keepdims=True)
    inv = lax.rsqrt(var + _EPS)
    o_ref[...] = xc * inv * g_ref[...] + b_ref[...]


def _tc_ln_alias_body(x_ref, tt_ref, pos_ref, type_ref, g_ref, b_ref,
                      acc_ref, o_ref):
    del acc_ref
    _tc_ln_body(x_ref, tt_ref, pos_ref, type_ref, g_ref, b_ref, o_ref)


def _tc_ln_slice(k, gathered, tt, pos_emb, type_emb, gamma2d, beta2d, acc):
    base_specs = [
        pl.BlockSpec((_S, _D), lambda b: (b, 0)),
        pl.BlockSpec((_S, 1), lambda b: (b, 0)),
        pl.BlockSpec((_S, _D), lambda b: (0, 0)),
        pl.BlockSpec((2, _D), lambda b: (0, 0)),
        pl.BlockSpec((1, _D), lambda b: (0, 0)),
        pl.BlockSpec((1, _D), lambda b: (0, 0)),
    ]
    out_spec = pl.BlockSpec((_S, _D), lambda b, k=k: (k * _SLICE_B + b, 0))
    args = (gathered, tt, pos_emb, type_emb, gamma2d, beta2d)
    if k == 0:
        return pl.pallas_call(
            _tc_ln_body,
            grid=(_SLICE_B,),
            in_specs=base_specs,
            out_specs=out_spec,
            out_shape=jax.ShapeDtypeStruct((_TOK, _D), jnp.float32),
        )(*args)
    return pl.pallas_call(
        _tc_ln_alias_body,
        grid=(_SLICE_B,),
        in_specs=base_specs + [pl.BlockSpec((8, 128), lambda b: (0, 0))],
        out_specs=out_spec,
        out_shape=jax.ShapeDtypeStruct((_TOK, _D), jnp.float32),
        input_output_aliases={6: 0},
    )(*args, acc)


def kernel(input_ids, token_type_ids, word_emb, pos_emb, type_emb, gamma, beta):
    ids = input_ids.astype(jnp.int32).reshape(_NSLICE, _NW, _NCH, _CH)
    tt = token_type_ids.reshape(_NSLICE, _SLICE_TOK, 1).astype(jnp.float32)
    gamma2d = gamma.reshape(1, _D)
    beta2d = beta.reshape(1, _D)
    out = None
    for k in range(_NSLICE):
        gathered = _sc_gather(ids[k], word_emb)
        out = _tc_ln_slice(k, gathered, tt[k], pos_emb, type_emb,
                           gamma2d, beta2d, out)
    return out.reshape(_B, _S, _D)


# speedup vs baseline: 1.0747x; 1.0163x over previous
import functools

import jax
import jax.numpy as jnp
from jax import lax
from jax.experimental import pallas as pl
from jax.experimental.pallas import tpu as pltpu
from jax.experimental.pallas import tpu_sc as plsc

_B, _S, _D = 64, 512, 768
_EPS = 1e-12

_NC, _NS = 2, 16
_NW = _NC * _NS
_TOK = _B * _S
_NSLICE = 4
_SLICE_TOK = _TOK // _NSLICE
_SLICE_B = _B // _NSLICE
_PER_W = _SLICE_TOK // _NW
_CH = 32
_NCH = _PER_W // _CH
_NBUF = 4
_LNB = 2 * _S


def _sc_gather_body(ids_hbm, table_hbm, out_hbm, idx_v, rows_v, *sems):
    wid = lax.axis_index("s") * _NC + lax.axis_index("c")
    base = wid * _PER_W
    pltpu.sync_copy(ids_hbm.at[wid], idx_v)

    gsems = sems[:_NBUF]
    wsems = sems[_NBUF:]

    def fire(c, b):
        return pltpu.async_copy(table_hbm.at[idx_v.at[c]], rows_v.at[b],
                                gsems[b])

    def drain(c, b):
        return pltpu.async_copy(rows_v.at[b],
                                out_hbm.at[pl.ds(base + c * _CH, _CH)],
                                wsems[b])

    gathers = [fire(b, b) for b in range(_NBUF)]
    drains = [None] * _NBUF
    for c in range(_NCH):
        b = c % _NBUF
        gathers[b].wait()
        drains[b] = drain(c, b)
        r = c + 2
        if c >= 2 and r < _NCH:
            rb = r % _NBUF
            drains[rb].wait()
            gathers[rb] = fire(r, rb)
    for b in range(_NBUF):
        drains[b].wait()


_sc_gather = functools.partial(
    pl.kernel,
    out_type=jax.ShapeDtypeStruct((_SLICE_TOK, _D), jnp.float32),
    mesh=plsc.VectorSubcoreMesh(
        core_axis_name="c", subcore_axis_name="s",
        num_cores=_NC, num_subcores=_NS),
    scratch_types=[
        pltpu.VMEM((_NCH, _CH), jnp.int32),
        pltpu.VMEM((_NBUF, _CH, _D), jnp.float32),
    ] + [pltpu.SemaphoreType.DMA] * (2 * _NBUF),
)(_sc_gather_body)


def _tc_ln_body(x_ref, tt_ref, pos_ref, type_ref, g_ref, b_ref, o_ref):
    x = x_ref[...] + pos_ref[...]
    t0 = type_ref[0:1, :]
    t1 = type_ref[1:2, :]
    tt = tt_ref[...]
    x = x + t0 + tt * (t1 - t0)
    mu = jnp.mean(x, axis=1, keepdims=True)
    xc = x - mu
    var = jnp.mean(xc * xc, axis=1, keepdims=True)
    inv = lax.rsqrt(var + _EPS)
    o_ref[...] = xc * inv * g_ref[...] + b_ref[...]


def _tc_ln_alias_body(x_ref, tt_ref, pos_ref, type_ref, g_ref, b_ref,
                      acc_ref, o_ref):
    del acc_ref
    _tc_ln_body(x_ref, tt_ref, pos_ref, type_ref, g_ref, b_ref, o_ref)


def _tc_ln_slice(k, gathered, tt, pos_emb, type_emb, gamma2d, beta2d, acc):
    nb = _LNB // _S
    base_specs = [
        pl.BlockSpec((_LNB, _D), lambda b: (b, 0)),
        pl.BlockSpec((_LNB, 1), lambda b: (b, 0)),
        pl.BlockSpec((_LNB, _D), lambda b: (0, 0)),
        pl.BlockSpec((2, _D), lambda b: (0, 0)),
        pl.BlockSpec((1, _D), lambda b: (0, 0)),
        pl.BlockSpec((1, _D), lambda b: (0, 0)),
    ]
    out_spec = pl.BlockSpec(
        (_LNB, _D), lambda b, k=k: (k * _SLICE_B // nb + b, 0))
    grid = (_SLICE_B // nb,)
    args = (gathered, tt, pos_emb, type_emb, gamma2d, beta2d)
    if k == 0:
        return pl.pallas_call(
            _tc_ln_body,
            grid=grid,
            in_specs=base_specs,
            out_specs=out_spec,
            out_shape=jax.ShapeDtypeStruct((_TOK, _D), jnp.float32),
        )(*args)
    return pl.pallas_call(
        _tc_ln_alias_body,
        grid=grid,
        in_specs=base_specs + [pl.BlockSpec((8, 128), lambda b: (0, 0))],
        out_specs=out_spec,
        out_shape=jax.ShapeDtypeStruct((_TOK, _D), jnp.float32),
        input_output_aliases={6: 0},
    )(*args, acc)


def kernel(input_ids, token_type_ids, word_emb, pos_emb, type_emb, gamma, beta):
    ids = input_ids.astype(jnp.int32).reshape(_NSLICE, _NW, _NCH, _CH)
    tt = token_type_ids.reshape(_NSLICE, _SLICE_TOK, 1).astype(jnp.float32)
    pos_emb = jnp.concatenate([pos_emb] * (_LNB // _S), axis=0)
    gamma2d = gamma.reshape(1, _D)
    beta2d = beta.reshape(1, _D)
    out = None
    for k in range(_NSLICE):
        gathered = _sc_gather(ids[k], word_emb)
        out = _tc_ln_slice(k, gathered, tt[k], pos_emb, type_emb,
                           gamma2d, beta2d, out)
    return out.reshape(_B, _S, _D)


# speedup vs baseline: 1.0907x; 1.0149x over previous
import functools

import jax
import jax.numpy as jnp
from jax import lax
from jax.experimental import pallas as pl
from jax.experimental.pallas import tpu as pltpu
from jax.experimental.pallas import tpu_sc as plsc

_B, _S, _D = 64, 512, 768
_EPS = 1e-12

_NC, _NS = 2, 16
_NW = _NC * _NS
_TOK = _B * _S
_NSLICE = 2
_SLICE_TOK = _TOK // _NSLICE
_SLICE_B = _B // _NSLICE
_PER_W = _SLICE_TOK // _NW
_CH = 32
_NCH = _PER_W // _CH
_NBUF = 4
_LNB = 2 * _S


def _sc_gather_body(ids_hbm, table_hbm, out_hbm, idx_v, rows_v, *sems):
    wid = lax.axis_index("s") * _NC + lax.axis_index("c")
    base = wid * _PER_W
    pltpu.sync_copy(ids_hbm.at[wid], idx_v)

    gsems = sems[:_NBUF]
    wsems = sems[_NBUF:]

    def fire(c, b):
        return pltpu.async_copy(table_hbm.at[idx_v.at[c]], rows_v.at[b],
                                gsems[b])

    def drain(c, b):
        return pltpu.async_copy(rows_v.at[b],
                                out_hbm.at[pl.ds(base + c * _CH, _CH)],
                                wsems[b])

    gathers = [fire(b, b) for b in range(_NBUF)]
    drains = [None] * _NBUF
    for c in range(_NCH):
        b = c % _NBUF
        gathers[b].wait()
        drains[b] = drain(c, b)
        r = c + 2
        if c >= 2 and r < _NCH:
            rb = r % _NBUF
            drains[rb].wait()
            gathers[rb] = fire(r, rb)
    for b in range(_NBUF):
        drains[b].wait()


_sc_gather = functools.partial(
    pl.kernel,
    out_type=jax.ShapeDtypeStruct((_SLICE_TOK, _D), jnp.float32),
    mesh=plsc.VectorSubcoreMesh(
        core_axis_name="c", subcore_axis_name="s",
        num_cores=_NC, num_subcores=_NS),
    scratch_types=[
        pltpu.VMEM((_NCH, _CH), jnp.int32),
        pltpu.VMEM((_NBUF, _CH, _D), jnp.float32),
    ] + [pltpu.SemaphoreType.DMA] * (2 * _NBUF),
)(_sc_gather_body)


def _tc_ln_body(x_ref, tt_ref, pos_ref, type_ref, g_ref, b_ref, o_ref):
    x = x_ref[...] + pos_ref[...]
    t0 = type_ref[0:1, :]
    t1 = type_ref[1:2, :]
    tt = tt_ref[...]
    x = x + t0 + tt * (t1 - t0)
    mu = jnp.mean(x, axis=1, keepdims=True)
    xc = x - mu
    var = jnp.mean(xc * xc, axis=1, keepdims=True)
    inv = lax.rsqrt(var + _EPS)
    o_ref[...] = xc * inv * g_ref[...] + b_ref[...]


def _tc_ln_alias_body(x_ref, tt_ref, pos_ref, type_ref, g_ref, b_ref,
                      acc_ref, o_ref):
    del acc_ref
    _tc_ln_body(x_ref, tt_ref, pos_ref, type_ref, g_ref, b_ref, o_ref)


def _tc_ln_slice(k, gathered, tt, pos_emb, type_emb, gamma2d, beta2d, acc):
    nb = _LNB // _S
    base_specs = [
        pl.BlockSpec((_LNB, _D), lambda b: (b, 0)),
        pl.BlockSpec((_LNB, 1), lambda b: (b, 0)),
        pl.BlockSpec((_LNB, _D), lambda b: (0, 0)),
        pl.BlockSpec((2, _D), lambda b: (0, 0)),
        pl.BlockSpec((1, _D), lambda b: (0, 0)),
        pl.BlockSpec((1, _D), lambda b: (0, 0)),
    ]
    out_spec = pl.BlockSpec(
        (_LNB, _D), lambda b, k=k: (k * _SLICE_B // nb + b, 0))
    grid = (_SLICE_B // nb,)
    args = (gathered, tt, pos_emb, type_emb, gamma2d, beta2d)
    if k == 0:
        return pl.pallas_call(
            _tc_ln_body,
            grid=grid,
            in_specs=base_specs,
            out_specs=out_spec,
            out_shape=jax.ShapeDtypeStruct((_TOK, _D), jnp.float32),
        )(*args)
    return pl.pallas_call(
        _tc_ln_alias_body,
        grid=grid,
        in_specs=base_specs + [pl.BlockSpec((8, 128), lambda b: (0, 0))],
        out_specs=out_spec,
        out_shape=jax.ShapeDtypeStruct((_TOK, _D), jnp.float32),
        input_output_aliases={6: 0},
    )(*args, acc)


def kernel(input_ids, token_type_ids, word_emb, pos_emb, type_emb, gamma, beta):
    ids = input_ids.astype(jnp.int32).reshape(_NSLICE, _NW, _NCH, _CH)
    tt = token_type_ids.reshape(_NSLICE, _SLICE_TOK, 1).astype(jnp.float32)
    pos_emb = jnp.concatenate([pos_emb] * (_LNB // _S), axis=0)
    gamma2d = gamma.reshape(1, _D)
    beta2d = beta.reshape(1, _D)
    out = None
    for k in range(_NSLICE):
        gathered = _sc_gather(ids[k], word_emb)
        out = _tc_ln_slice(k, gathered, tt[k], pos_emb, type_emb,
                           gamma2d, beta2d, out)
    return out.reshape(_B, _S, _D)
